# baseline (speedup 1.0000x reference)
import jax
import jax.numpy as jnp
from jax.experimental import pallas as pl
from jax.experimental.pallas import tpu as pltpu

H = 1024; I = 4096; MI = 512; E = 8; K = 2; EPS = 1e-06; B = 1; S = 2048
T = B * S
TT = 256
IT = 512


def _stage_a_body(x_ref, plw_ref, pl2w_ref, rwt_ref, rs_ref, pes_ref,
                  h1_ref, h2_ref, aff_ref, ti_ref, tw_ref):
    x = x_ref[...]
    ms = jnp.mean(x * x, axis=1, keepdims=True) + EPS
    xn = x * jax.lax.rsqrt(ms)
    h1_ref[...] = (xn * plw_ref[...]).astype(jnp.bfloat16)
    h2_ref[...] = (xn * pl2w_ref[...]).astype(jnp.bfloat16)
    xr = xn * rs_ref[...] * (H ** -0.5)
    logits = jnp.dot(xr, rwt_ref[...], preferred_element_type=jnp.float32,
                     precision=jax.lax.Precision.DEFAULT)
    m = jnp.max(logits, axis=1, keepdims=True)
    p = jnp.exp(logits - m)
    probs = p / jnp.sum(p, axis=1, keepdims=True)
    iota = jax.lax.broadcasted_iota(jnp.int32, probs.shape, 1)
    m1 = jnp.max(probs, axis=1, keepdims=True)
    i1 = jnp.min(jnp.where(probs == m1, iota, E), axis=1, keepdims=True)
    probs2 = jnp.where(iota == i1, -jnp.inf, probs)
    m2 = jnp.max(probs2, axis=1, keepdims=True)
    i2 = jnp.min(jnp.where(probs2 == m2, iota, E), axis=1, keepdims=True)
    s = m1 + m2
    pes = pes_ref[...]
    pes1 = jnp.sum(jnp.where(iota == i1, pes, 0.0), axis=1, keepdims=True)
    pes2 = jnp.sum(jnp.where(iota == i2, pes, 0.0), axis=1, keepdims=True)
    w1 = m1 / s * pes1
    w2 = m2 / s * pes2
    aff_ref[...] = jnp.where(iota == i1, w1, 0.0) + jnp.where(iota == i2, w2, 0.0)
    ti_ref[...] = jnp.concatenate([i1, i2], axis=1)
    tw_ref[...] = jnp.concatenate([w1, w2], axis=1)


def _glu_body(h1_ref, gt_ref, ut_ref, dt_ref, out_ref):
    i = pl.program_id(0)
    h1 = h1_ref[...]
    g = jnp.dot(h1, gt_ref[...], preferred_element_type=jnp.float32)
    u = jnp.dot(h1, ut_ref[...], preferred_element_type=jnp.float32)
    hm = (jax.nn.gelu(g, approximate=True) * u).astype(jnp.bfloat16)
    contrib = jnp.dot(hm, dt_ref[...], preferred_element_type=jnp.float32)

    @pl.when(i == 0)
    def _():
        out_ref[...] = contrib

    @pl.when(i > 0)
    def _():
        out_ref[...] += contrib


def _moe_dense_body(h2_ref, aff_ref, egt_ref, eut_ref, edt_ref, out_ref):
    e = pl.program_id(0)
    h2 = h2_ref[...]
    g = jnp.dot(h2, egt_ref[0], preferred_element_type=jnp.float32)
    u = jnp.dot(h2, eut_ref[0], preferred_element_type=jnp.float32)
    hm = (jax.nn.gelu(g, approximate=True) * u).astype(jnp.bfloat16)
    o = jnp.dot(hm, edt_ref[0], preferred_element_type=jnp.float32)
    aff = aff_ref[...]
    eiota = jax.lax.broadcasted_iota(jnp.int32, aff.shape, 1)
    w = jnp.sum(jnp.where(eiota == e, aff, 0.0), axis=1, keepdims=True)
    contrib = w * o

    @pl.when(e == 0)
    def _():
        out_ref[...] = contrib

    @pl.when(e > 0)
    def _():
        out_ref[...] += contrib


def _final_body(mlp_ref, moe_ref, p1_ref, p2_ref, pf_ref, out_ref):
    def rms(v, w):
        ms = jnp.mean(v * v, axis=1, keepdims=True) + EPS
        return v * jax.lax.rsqrt(ms) * w

    a = rms(mlp_ref[...], p1_ref[...])
    b = rms(moe_ref[...], p2_ref[...])
    out_ref[...] = rms(a + b, pf_ref[...])


def kernel(hidden_states, pre_ln_w, pre_ln2_w, post_ln1_w, post_ln2_w,
           post_ln_w, gate_w, up_w, down_w, router_w, router_scale,
           per_expert_scale, exp_gate_w, exp_up_w, exp_down_w):
    bf16 = jnp.bfloat16
    x2d = hidden_states.reshape(T, H)
    plw = pre_ln_w.reshape(1, H)
    pl2w = pre_ln2_w.reshape(1, H)
    rs = router_scale.reshape(1, H)
    rwt = router_w.T
    pes = per_expert_scale.reshape(1, E)
    gate_t = gate_w.T.astype(bf16)
    up_t = up_w.T.astype(bf16)
    down_t = down_w.T.astype(bf16)
    egt = exp_gate_w.transpose(0, 2, 1).astype(bf16)
    eut = exp_up_w.transpose(0, 2, 1).astype(bf16)
    edt = exp_down_w.transpose(0, 2, 1).astype(bf16)

    n_tt = T // TT
    h1, h2, aff, ti, tw = pl.pallas_call(
        _stage_a_body,
        grid=(n_tt,),
        in_specs=[
            pl.BlockSpec((TT, H), lambda i: (i, 0)),
            pl.BlockSpec((1, H), lambda i: (0, 0)),
            pl.BlockSpec((1, H), lambda i: (0, 0)),
            pl.BlockSpec((H, E), lambda i: (0, 0)),
            pl.BlockSpec((1, H), lambda i: (0, 0)),
            pl.BlockSpec((1, E), lambda i: (0, 0)),
        ],
        out_specs=[
            pl.BlockSpec((TT, H), lambda i: (i, 0)),
            pl.BlockSpec((TT, H), lambda i: (i, 0)),
            pl.BlockSpec((TT, E), lambda i: (i, 0)),
            pl.BlockSpec((TT, K), lambda i: (i, 0)),
            pl.BlockSpec((TT, K), lambda i: (i, 0)),
        ],
        out_shape=[
            jax.ShapeDtypeStruct((T, H), bf16),
            jax.ShapeDtypeStruct((T, H), bf16),
            jax.ShapeDtypeStruct((T, E), jnp.float32),
            jax.ShapeDtypeStruct((T, K), jnp.int32),
            jax.ShapeDtypeStruct((T, K), jnp.float32),
        ],
    )(x2d, plw, pl2w, rwt, rs, pes)

    mlp_raw = pl.pallas_call(
        _glu_body,
        grid=(I // IT,),
        in_specs=[
            pl.BlockSpec((T, H), lambda i: (0, 0)),
            pl.BlockSpec((H, IT), lambda i: (0, i)),
            pl.BlockSpec((H, IT), lambda i: (0, i)),
            pl.BlockSpec((IT, H), lambda i: (i, 0)),
        ],
        out_specs=pl.BlockSpec((T, H), lambda i: (0, 0)),
        out_shape=jax.ShapeDtypeStruct((T, H), jnp.float32),
    )(h1, gate_t, up_t, down_t)

    moe_raw = pl.pallas_call(
        _moe_dense_body,
        grid=(E,),
        in_specs=[
            pl.BlockSpec((T, H), lambda e: (0, 0)),
            pl.BlockSpec((T, E), lambda e: (0, 0)),
            pl.BlockSpec((1, H, MI), lambda e: (e, 0, 0)),
            pl.BlockSpec((1, H, MI), lambda e: (e, 0, 0)),
            pl.BlockSpec((1, MI, H), lambda e: (e, 0, 0)),
        ],
        out_specs=pl.BlockSpec((T, H), lambda e: (0, 0)),
        out_shape=jax.ShapeDtypeStruct((T, H), jnp.float32),
    )(h2, aff, egt, eut, edt)

    out = pl.pallas_call(
        _final_body,
        grid=(n_tt,),
        in_specs=[
            pl.BlockSpec((TT, H), lambda i: (i, 0)),
            pl.BlockSpec((TT, H), lambda i: (i, 0)),
            pl.BlockSpec((1, H), lambda i: (0, 0)),
            pl.BlockSpec((1, H), lambda i: (0, 0)),
            pl.BlockSpec((1, H), lambda i: (0, 0)),
        ],
        out_specs=pl.BlockSpec((TT, H), lambda i: (i, 0)),
        out_shape=jax.ShapeDtypeStruct((T, H), jnp.float32),
    )(mlp_raw, moe_raw, post_ln1_w.reshape(1, H), post_ln2_w.reshape(1, H),
      post_ln_w.reshape(1, H))

    return out.reshape(B, S, H)
